# Initial kernel scaffold; baseline (speedup 1.0000x reference)
import jax
import jax.numpy as jnp
from jax.experimental import pallas as pl


def kernel(connectivity, nodes_data, W, attn_l, attn_r, bias):
    raise NotImplementedError("write your pallas kernel here")


# speedup vs baseline: 33.8429x; 33.8429x over previous
import functools

import jax
import jax.numpy as jnp
from jax import lax
from jax.experimental import pallas as pl
from jax.experimental.pallas import tpu as pltpu
from jax.experimental.pallas import tpu_sc as plsc

_N = 10000
_IN = 256
_H = 8
_D = 32
_E = 320000
_HD = _H * _D
_HALF = _HD // 2
_HH = _H // 2
_LP = 16

_BN = 400
_B = 80
_NS = 16
_NC = 2
_CHUNK = _E // _NS
_NBLK = _CHUNK // _B
_RPS = 632
_RPS_LAST = _N - 15 * _RPS


def _proj_body(x_ref, w_ref, al_ref, ar_ref, feat_ref, el_ref, er_ref):
    x = x_ref[...]
    w = w_ref[...]
    feat = jnp.dot(x, w, preferred_element_type=jnp.float32)
    feat_ref[...] = feat
    f3 = feat.reshape(_BN, _H, _D)
    el = jnp.sum(f3 * al_ref[...][None, :, :], axis=-1)
    er = jnp.sum(f3 * ar_ref[...][None, :, :], axis=-1)
    pad = jnp.zeros((_BN, _LP - _H), jnp.float32)
    el_ref[...] = jnp.concatenate([el, pad], axis=1)
    er_ref[...] = jnp.concatenate([er, pad], axis=1)


def _project(x, w, al, ar):
    grid = _N // _BN
    return pl.pallas_call(
        _proj_body,
        grid=(grid,),
        in_specs=[
            pl.BlockSpec((_BN, _IN), lambda i: (i, 0)),
            pl.BlockSpec((_IN, _HD), lambda i: (0, 0)),
            pl.BlockSpec((_H, _D), lambda i: (0, 0)),
            pl.BlockSpec((_H, _D), lambda i: (0, 0)),
        ],
        out_specs=[
            pl.BlockSpec((_BN, _HD), lambda i: (i, 0)),
            pl.BlockSpec((_BN, _LP), lambda i: (i, 0)),
            pl.BlockSpec((_BN, _LP), lambda i: (i, 0)),
        ],
        out_shape=[
            jax.ShapeDtypeStruct((_N, _HD), jnp.float32),
            jax.ShapeDtypeStruct((_N, _LP), jnp.float32),
            jax.ShapeDtypeStruct((_N, _LP), jnp.float32),
        ],
    )(x, w, al, ar)


def _sc_body(src_hbm, dst_hbm, feat2_hbm, el_hbm, er_hbm, zrow_hbm, zden_hbm,
             acc_hbm, den_hbm,
             idx_s, idx_d, idx2, elg, erg, eev, eef, rows,
             sem_a, sem_b, sem_c,
             acc_sh, den_sh):
    c = lax.axis_index("c")
    s = lax.axis_index("s")

    @pl.when(s < 15)
    def _zero_a():
        pltpu.sync_copy(zrow_hbm, acc_sh.at[pl.ds(s * _RPS, _RPS)])

        @pl.when(c == 0)
        def _zden_a():
            pltpu.sync_copy(zden_hbm, den_sh.at[pl.ds(s * _RPS, _RPS)])

    @pl.when(s == 15)
    def _zero_b():
        pltpu.sync_copy(zrow_hbm.at[pl.ds(0, _RPS_LAST)],
                        acc_sh.at[pl.ds(15 * _RPS, _RPS_LAST)])

        @pl.when(c == 0)
        def _zden_b():
            pltpu.sync_copy(zden_hbm.at[pl.ds(0, _RPS_LAST)],
                            den_sh.at[pl.ds(15 * _RPS, _RPS_LAST)])

    plsc.subcore_barrier()

    base_e = s * _CHUNK
    hbase = c * _HH

    def block(b, carry):
        e0 = base_e + b * _B
        pltpu.sync_copy(src_hbm.at[pl.ds(e0, _B)], idx_s)
        pltpu.sync_copy(dst_hbm.at[pl.ds(e0, _B)], idx_d)

        def mk(k, carry2):
            v = idx_s[pl.ds(k * 16, 16)]
            idx2[pl.ds(k * 16, 16)] = v * 2 + c
            return carry2

        lax.fori_loop(0, _B // 16, mk, 0)

        cp_el = pltpu.async_copy(el_hbm.at[idx_s], elg, sem_a)
        cp_er = pltpu.async_copy(er_hbm.at[idx_d], erg, sem_b)
        cp_f = pltpu.async_copy(feat2_hbm.at[idx2], rows, sem_c)
        cp_el.wait()
        cp_er.wait()

        def ecomp(k, carry2):
            a = elg[k, :] + erg[k, :]
            a = jnp.maximum(a, a * 0.2)
            a = jnp.exp(a)
            eev[k, :] = a
            eef[pl.ds(k * 16, 16)] = a
            return carry2

        lax.fori_loop(0, _B, ecomp, 0)

        @pl.when(c == 0)
        def _den_add():
            pltpu.sync_copy(eev, den_sh.at[idx_d], add=True)

        cp_f.wait()

        def scale(e, carry2):
            eb = e * _LP + hbase
            for h in range(_HH):
                bidx = jnp.zeros((16,), jnp.int32) + (eb + h)
                bv = plsc.load_gather(eef, [bidx])
                for j2 in range(2):
                    j = (h * 2 + j2) * 16
                    rows[e, pl.ds(j, 16)] = rows[e, pl.ds(j, 16)] * bv
            return carry2

        lax.fori_loop(0, _B, scale, 0)

        pltpu.sync_copy(rows, acc_sh.at[idx_d], add=True)
        return carry

    lax.fori_loop(0, _NBLK, block, 0)
    plsc.subcore_barrier()

    @pl.when(s < 15)
    def _out_a():
        pltpu.sync_copy(acc_sh.at[pl.ds(s * _RPS, _RPS)],
                        acc_hbm.at[pl.ds(c * _N + s * _RPS, _RPS)])

        @pl.when(c == 0)
        def _dout_a():
            pltpu.sync_copy(den_sh.at[pl.ds(s * _RPS, _RPS)],
                            den_hbm.at[pl.ds(s * _RPS, _RPS)])

    @pl.when(s == 15)
    def _out_b():
        pltpu.sync_copy(acc_sh.at[pl.ds(15 * _RPS, _RPS_LAST)],
                        acc_hbm.at[pl.ds(c * _N + 15 * _RPS, _RPS_LAST)])

        @pl.when(c == 0)
        def _dout_b():
            pltpu.sync_copy(den_sh.at[pl.ds(15 * _RPS, _RPS_LAST)],
                            den_hbm.at[pl.ds(15 * _RPS, _RPS_LAST)])


def _edge_phase(src, dst, feat2, el, er):
    zrow = jnp.zeros((_RPS, _HALF), jnp.float32)
    zden = jnp.zeros((_RPS, _LP), jnp.float32)
    mesh = plsc.VectorSubcoreMesh(core_axis_name="c", subcore_axis_name="s")
    call = pl.kernel(
        _sc_body,
        out_type=[
            jax.ShapeDtypeStruct((_NC * _N, _HALF), jnp.float32),
            jax.ShapeDtypeStruct((_N, _LP), jnp.float32),
        ],
        mesh=mesh,
        compiler_params=pltpu.CompilerParams(
            needs_layout_passes=False, use_tc_tiling_on_sc=False),
        scratch_types=[
            pltpu.VMEM((_B,), jnp.int32),
            pltpu.VMEM((_B,), jnp.int32),
            pltpu.VMEM((_B,), jnp.int32),
            pltpu.VMEM((_B, _LP), jnp.float32),
            pltpu.VMEM((_B, _LP), jnp.float32),
            pltpu.VMEM((_B, _LP), jnp.float32),
            pltpu.VMEM((_B * _LP,), jnp.float32),
            pltpu.VMEM((_B, _HALF), jnp.float32),
            pltpu.SemaphoreType.DMA,
            pltpu.SemaphoreType.DMA,
            pltpu.SemaphoreType.DMA,
            pltpu.VMEM_SHARED((_N, _HALF), jnp.float32),
            pltpu.VMEM_SHARED((_N, _LP), jnp.float32),
        ],
    )
    return call(src, dst, feat2, el, er, zrow, zden)


def _epi_body(lo_ref, hi_ref, den_ref, bias_ref, out_ref):
    cat = jnp.concatenate([lo_ref[...], hi_ref[...]], axis=1)
    den = jnp.maximum(den_ref[...][:, :_H], 1e-9)
    den_exp = jnp.broadcast_to(den[:, :, None], (_BN, _H, _D))
    out_ref[...] = cat / den_exp.reshape(_BN, _HD) + bias_ref[...][None, :]


def _epilogue(acc, den, bias):
    grid = _N // _BN
    nb = _N // _BN
    return pl.pallas_call(
        _epi_body,
        grid=(grid,),
        in_specs=[
            pl.BlockSpec((_BN, _HALF), lambda i: (i, 0)),
            pl.BlockSpec((_BN, _HALF), lambda i, _nb=nb: (i + _nb, 0)),
            pl.BlockSpec((_BN, _LP), lambda i: (i, 0)),
            pl.BlockSpec((_HD,), lambda i: (0,)),
        ],
        out_specs=pl.BlockSpec((_BN, _HD), lambda i: (i, 0)),
        out_shape=jax.ShapeDtypeStruct((_N, _HD), jnp.float32),
    )(acc, acc, den, bias)


def kernel(connectivity, nodes_data, W, attn_l, attn_r, bias):
    src = connectivity[0, 0]
    dst = connectivity[1, 0]
    feat, el, er = _project(nodes_data, W, attn_l, attn_r)
    feat2 = feat.reshape(_NC * _N, _HALF)
    acc, den = _edge_phase(src, dst, feat2, el, er)
    return _epilogue(acc, den, bias)


# speedup vs baseline: 50.7251x; 1.4988x over previous
import functools

import jax
import jax.numpy as jnp
from jax import lax
from jax.experimental import pallas as pl
from jax.experimental.pallas import tpu as pltpu
from jax.experimental.pallas import tpu_sc as plsc

_N = 10000
_IN = 256
_H = 8
_D = 32
_E = 320000
_HD = _H * _D
_HALF = _HD // 2
_HH = _H // 2
_LP = 16

_BN = 400
_B = 80
_NS = 16
_NC = 2
_CHUNK = _E // _NS
_NBLK = _CHUNK // _B
_RPS = 632
_RPS_LAST = _N - 15 * _RPS


def _proj_body(x_ref, w_ref, al_ref, ar_ref, feat_ref, el_ref, er_ref):
    x = x_ref[...]
    w = w_ref[...]
    feat = jnp.dot(x, w, preferred_element_type=jnp.float32)
    feat_ref[...] = feat
    f3 = feat.reshape(_BN, _H, _D)
    el = jnp.sum(f3 * al_ref[...][None, :, :], axis=-1)
    er = jnp.sum(f3 * ar_ref[...][None, :, :], axis=-1)
    pad = jnp.zeros((_BN, _LP - _H), jnp.float32)
    el_ref[...] = jnp.concatenate([el, pad], axis=1)
    er_ref[...] = jnp.concatenate([er, pad], axis=1)


def _project(x, w, al, ar):
    grid = _N // _BN
    return pl.pallas_call(
        _proj_body,
        grid=(grid,),
        in_specs=[
            pl.BlockSpec((_BN, _IN), lambda i: (i, 0)),
            pl.BlockSpec((_IN, _HD), lambda i: (0, 0)),
            pl.BlockSpec((_H, _D), lambda i: (0, 0)),
            pl.BlockSpec((_H, _D), lambda i: (0, 0)),
        ],
        out_specs=[
            pl.BlockSpec((_BN, _HD), lambda i: (i, 0)),
            pl.BlockSpec((_BN, _LP), lambda i: (i, 0)),
            pl.BlockSpec((_BN, _LP), lambda i: (i, 0)),
        ],
        out_shape=[
            jax.ShapeDtypeStruct((_N, _HD), jnp.float32),
            jax.ShapeDtypeStruct((_N, _LP), jnp.float32),
            jax.ShapeDtypeStruct((_N, _LP), jnp.float32),
        ],
    )(x, w, al, ar)


def _sc_body(src_hbm, dst_hbm, feat2_hbm, el_hbm, er_hbm, zrow_hbm, zden_hbm,
             acc_hbm, den_hbm,
             idx_sa, idx_sb, idx_da, idx_db, sidxa, sidxb,
             idx2a, idx2b, elga, elgb, erga, ergb, eeva, eevb, eefa, eefb,
             rowsa, rowsb,
             sem_is_a, sem_is_b, sem_id_a, sem_id_b,
             sem_el_a, sem_el_b, sem_er_a, sem_er_b, sem_f_a, sem_f_b,
             acc_sh, den_sh):
    c = lax.axis_index("c")
    s = lax.axis_index("s")
    idx_s_ = (idx_sa, idx_sb)
    idx_d_ = (idx_da, idx_db)
    sidx_ = (sidxa, sidxb)
    sem_is = (sem_is_a, sem_is_b)
    sem_id = (sem_id_a, sem_id_b)
    idx2_ = (idx2a, idx2b)
    elg_ = (elga, elgb)
    erg_ = (erga, ergb)
    eev_ = (eeva, eevb)
    eef_ = (eefa, eefb)
    rows_ = (rowsa, rowsb)
    sem_el = (sem_el_a, sem_el_b)
    sem_er = (sem_er_a, sem_er_b)
    sem_f = (sem_f_a, sem_f_b)

    @pl.when(s < 15)
    def _zero_a():
        pltpu.sync_copy(zrow_hbm, acc_sh.at[pl.ds(s * _RPS, _RPS)])

        @pl.when(c == 0)
        def _zden_a():
            pltpu.sync_copy(zden_hbm, den_sh.at[pl.ds(s * _RPS, _RPS)])

    @pl.when(s == 15)
    def _zero_b():
        pltpu.sync_copy(zrow_hbm.at[pl.ds(0, _RPS_LAST)],
                        acc_sh.at[pl.ds(15 * _RPS, _RPS_LAST)])

        @pl.when(c == 0)
        def _zden_b():
            pltpu.sync_copy(zden_hbm.at[pl.ds(0, _RPS_LAST)],
                            den_sh.at[pl.ds(15 * _RPS, _RPS_LAST)])

    plsc.subcore_barrier()

    hbase = c * _HH

    def idx_loads(b, slot):
        return (
            pltpu.make_async_copy(src_hbm.at[s, b], idx_s_[slot],
                                  sem_is[slot]),
            pltpu.make_async_copy(dst_hbm.at[s, b], idx_d_[slot],
                                  sem_id[slot]),
        )

    def gathers(b, slot):
        return (
            pltpu.make_async_copy(el_hbm.at[idx_s_[slot]], elg_[slot],
                                  sem_el[slot]),
            pltpu.make_async_copy(er_hbm.at[idx_d_[slot]], erg_[slot],
                                  sem_er[slot]),
            pltpu.make_async_copy(feat2_hbm.at[idx2_[slot]], rows_[slot],
                                  sem_f[slot]),
        )

    def start_idx(b, slot):
        for cp in idx_loads(b, slot):
            cp.start()

    def issue(b, slot):
        for cp in idx_loads(b, slot):
            cp.wait()

        def mk(k, carry2):
            v = idx_s_[slot][pl.ds(k * 16, 16)]
            idx2_[slot][pl.ds(k * 16, 16)] = v * 2 + c
            return carry2

        lax.fori_loop(0, _B // 16, mk, 0)
        for cp in gathers(b, slot):
            cp.start()

    def process(b, slot):
        cp_el, cp_er, cp_f = gathers(b, slot)
        cp_el.wait()
        cp_er.wait()

        elg = elg_[slot]
        erg = erg_[slot]
        eev = eev_[slot]
        eef = eef_[slot]
        rows = rows_[slot]
        sidx = sidx_[slot]
        idx_d = idx_d_[slot]

        def ecomp(k, carry2):
            a = elg[k, :] + erg[k, :]
            a = jnp.maximum(a, a * 0.2)
            a = jnp.exp(a)
            eev[k, :] = a
            eef[pl.ds(k * 16, 16)] = a
            return carry2

        lax.fori_loop(0, _B, ecomp, 0)

        def icopy(k, carry2):
            sidx[pl.ds(k * 16, 16)] = idx_d[pl.ds(k * 16, 16)]
            return carry2

        lax.fori_loop(0, _B // 16, icopy, 0)

        @pl.when(c == 0)
        def _den_add():
            pltpu.sync_copy(eev, den_sh.at[sidx], add=True)

        cp_f.wait()

        @pl.when(b + 2 < _NBLK)
        def _prefetch_idx():
            start_idx(b + 2, slot)

        def scale(e, carry2):
            eb = e * _LP + hbase
            for h in range(_HH):
                bidx = jnp.zeros((16,), jnp.int32) + (eb + h)
                bv = plsc.load_gather(eef, [bidx])
                for j2 in range(2):
                    j = (h * 2 + j2) * 16
                    rows[e, pl.ds(j, 16)] = rows[e, pl.ds(j, 16)] * bv
            return carry2

        lax.fori_loop(0, _B, scale, 0)

        pltpu.sync_copy(rows, acc_sh.at[sidx], add=True)

    start_idx(0, 0)
    issue(0, 0)
    start_idx(1, 1)

    def pair(g2, carry):
        b0 = 2 * g2
        b1 = b0 + 1
        issue(b1, 1)
        process(b0, 0)

        @pl.when(b0 + 2 < _NBLK)
        def _gather0():
            issue(b0 + 2, 0)

        process(b1, 1)
        return carry

    lax.fori_loop(0, _NBLK // 2, pair, 0)
    plsc.subcore_barrier()

    @pl.when(s < 15)
    def _out_a():
        pltpu.sync_copy(acc_sh.at[pl.ds(s * _RPS, _RPS)],
                        acc_hbm.at[pl.ds(c * _N + s * _RPS, _RPS)])

        @pl.when(c == 0)
        def _dout_a():
            pltpu.sync_copy(den_sh.at[pl.ds(s * _RPS, _RPS)],
                            den_hbm.at[pl.ds(s * _RPS, _RPS)])

    @pl.when(s == 15)
    def _out_b():
        pltpu.sync_copy(acc_sh.at[pl.ds(15 * _RPS, _RPS_LAST)],
                        acc_hbm.at[pl.ds(c * _N + 15 * _RPS, _RPS_LAST)])

        @pl.when(c == 0)
        def _dout_b():
            pltpu.sync_copy(den_sh.at[pl.ds(15 * _RPS, _RPS_LAST)],
                            den_hbm.at[pl.ds(15 * _RPS, _RPS_LAST)])


def _edge_phase(src, dst, feat2, el, er):
    zrow = jnp.zeros((_RPS, _HALF), jnp.float32)
    zden = jnp.zeros((_RPS, _LP), jnp.float32)
    mesh = plsc.VectorSubcoreMesh(core_axis_name="c", subcore_axis_name="s")
    call = pl.kernel(
        _sc_body,
        out_type=[
            jax.ShapeDtypeStruct((_NC * _N, _HALF), jnp.float32),
            jax.ShapeDtypeStruct((_N, _LP), jnp.float32),
        ],
        mesh=mesh,
        compiler_params=pltpu.CompilerParams(
            needs_layout_passes=False, use_tc_tiling_on_sc=False),
        scratch_types=[
            pltpu.VMEM((_B,), jnp.int32),
            pltpu.VMEM((_B,), jnp.int32),
            pltpu.VMEM((_B,), jnp.int32),
            pltpu.VMEM((_B,), jnp.int32),
            pltpu.VMEM((_B,), jnp.int32),
            pltpu.VMEM((_B,), jnp.int32),
            pltpu.VMEM((_B,), jnp.int32),
            pltpu.VMEM((_B,), jnp.int32),
            pltpu.VMEM((_B, _LP), jnp.float32),
            pltpu.VMEM((_B, _LP), jnp.float32),
            pltpu.VMEM((_B, _LP), jnp.float32),
            pltpu.VMEM((_B, _LP), jnp.float32),
            pltpu.VMEM((_B, _LP), jnp.float32),
            pltpu.VMEM((_B, _LP), jnp.float32),
            pltpu.VMEM((_B * _LP,), jnp.float32),
            pltpu.VMEM((_B * _LP,), jnp.float32),
            pltpu.VMEM((_B, _HALF), jnp.float32),
            pltpu.VMEM((_B, _HALF), jnp.float32),
            pltpu.SemaphoreType.DMA,
            pltpu.SemaphoreType.DMA,
            pltpu.SemaphoreType.DMA,
            pltpu.SemaphoreType.DMA,
            pltpu.SemaphoreType.DMA,
            pltpu.SemaphoreType.DMA,
            pltpu.SemaphoreType.DMA,
            pltpu.SemaphoreType.DMA,
            pltpu.SemaphoreType.DMA,
            pltpu.SemaphoreType.DMA,
            pltpu.VMEM_SHARED((_N, _HALF), jnp.float32),
            pltpu.VMEM_SHARED((_N, _LP), jnp.float32),
        ],
    )
    return call(src, dst, feat2, el, er, zrow, zden)


def _epi_body(lo_ref, hi_ref, den_ref, bias_ref, out_ref):
    cat = jnp.concatenate([lo_ref[...], hi_ref[...]], axis=1)
    den = jnp.maximum(den_ref[...][:, :_H], 1e-9)
    den_exp = jnp.broadcast_to(den[:, :, None], (_BN, _H, _D))
    out_ref[...] = cat / den_exp.reshape(_BN, _HD) + bias_ref[...][None, :]


def _epilogue(acc, den, bias):
    grid = _N // _BN
    nb = _N // _BN
    return pl.pallas_call(
        _epi_body,
        grid=(grid,),
        in_specs=[
            pl.BlockSpec((_BN, _HALF), lambda i: (i, 0)),
            pl.BlockSpec((_BN, _HALF), lambda i, _nb=nb: (i + _nb, 0)),
            pl.BlockSpec((_BN, _LP), lambda i: (i, 0)),
            pl.BlockSpec((_HD,), lambda i: (0,)),
        ],
        out_specs=pl.BlockSpec((_BN, _HD), lambda i: (i, 0)),
        out_shape=jax.ShapeDtypeStruct((_N, _HD), jnp.float32),
    )(acc, acc, den, bias)


def kernel(connectivity, nodes_data, W, attn_l, attn_r, bias):
    src = connectivity[0, 0].reshape(_NS, _NBLK, _B)
    dst = connectivity[1, 0].reshape(_NS, _NBLK, _B)
    feat, el, er = _project(nodes_data, W, attn_l, attn_r)
    feat2 = feat.reshape(_NC * _N, _HALF)
    acc, den = _edge_phase(src, dst, feat2, el, er)
    return _epilogue(acc, den, bias)


# speedup vs baseline: 92.4421x; 1.8224x over previous
import functools

import jax
import jax.numpy as jnp
from jax import lax
from jax.experimental import pallas as pl
from jax.experimental.pallas import tpu as pltpu
from jax.experimental.pallas import tpu_sc as plsc

_N = 10000
_IN = 256
_H = 8
_D = 32
_E = 320000
_HD = _H * _D
_HALF = _HD // 2
_HH = _H // 2
_LP = 16

_BN = 400
_B = 80
_NS = 16
_NC = 2
_CHUNK = _E // _NS
_NBLK = _CHUNK // _B
_RPS = 632
_RPS_LAST = _N - 15 * _RPS


def _proj_body(x_ref, w_ref, al_ref, ar_ref, feat_ref, el_ref, er_ref):
    x = x_ref[...]
    w = w_ref[...]
    feat = jnp.dot(x, w, preferred_element_type=jnp.float32)
    feat_ref[...] = feat
    f3 = feat.reshape(_BN, _H, _D)
    el = jnp.sum(f3 * al_ref[...][None, :, :], axis=-1)
    er = jnp.sum(f3 * ar_ref[...][None, :, :], axis=-1)
    pad = jnp.zeros((_BN, _LP - _H), jnp.float32)
    el_ref[...] = jnp.concatenate([el, pad], axis=1)
    er_ref[...] = jnp.concatenate([er, pad], axis=1)


def _project(x, w, al, ar):
    grid = _N // _BN
    return pl.pallas_call(
        _proj_body,
        grid=(grid,),
        in_specs=[
            pl.BlockSpec((_BN, _IN), lambda i: (i, 0)),
            pl.BlockSpec((_IN, _HD), lambda i: (0, 0)),
            pl.BlockSpec((_H, _D), lambda i: (0, 0)),
            pl.BlockSpec((_H, _D), lambda i: (0, 0)),
        ],
        out_specs=[
            pl.BlockSpec((_BN, _HD), lambda i: (i, 0)),
            pl.BlockSpec((_BN, _LP), lambda i: (i, 0)),
            pl.BlockSpec((_BN, _LP), lambda i: (i, 0)),
        ],
        out_shape=[
            jax.ShapeDtypeStruct((_N, _HD), jnp.float32),
            jax.ShapeDtypeStruct((_N, _LP), jnp.float32),
            jax.ShapeDtypeStruct((_N, _LP), jnp.float32),
        ],
    )(x, w, al, ar)


def _sc_body(src_hbm, dst_hbm, feat2_hbm, el_hbm, er_hbm, zrow_hbm, zden_hbm,
             acc_hbm, den_hbm,
             idx_sa, idx_sb, idx_da, idx_db, sidxa, sidxb,
             idx2a, idx2b, elga, elgb, erga, ergb, eeva, eevb, eefa, eefb,
             rowsa, rowsb,
             sem_is_a, sem_is_b, sem_id_a, sem_id_b,
             sem_el_a, sem_el_b, sem_er_a, sem_er_b, sem_f_a, sem_f_b,
             acc_sh, den_sh):
    c = lax.axis_index("c")
    s = lax.axis_index("s")
    idx_s_ = (idx_sa, idx_sb)
    idx_d_ = (idx_da, idx_db)
    sidx_ = (sidxa, sidxb)
    sem_is = (sem_is_a, sem_is_b)
    sem_id = (sem_id_a, sem_id_b)
    idx2_ = (idx2a, idx2b)
    elg_ = (elga, elgb)
    erg_ = (erga, ergb)
    eev_ = (eeva, eevb)
    eef_ = (eefa, eefb)
    rows_ = (rowsa, rowsb)
    sem_el = (sem_el_a, sem_el_b)
    sem_er = (sem_er_a, sem_er_b)
    sem_f = (sem_f_a, sem_f_b)

    @pl.when(s < 15)
    def _zero_a():
        pltpu.sync_copy(zrow_hbm, acc_sh.at[pl.ds(s * _RPS, _RPS)])

        @pl.when(c == 0)
        def _zden_a():
            pltpu.sync_copy(zden_hbm, den_sh.at[pl.ds(s * _RPS, _RPS)])

    @pl.when(s == 15)
    def _zero_b():
        pltpu.sync_copy(zrow_hbm.at[pl.ds(0, _RPS_LAST)],
                        acc_sh.at[pl.ds(15 * _RPS, _RPS_LAST)])

        @pl.when(c == 0)
        def _zden_b():
            pltpu.sync_copy(zden_hbm.at[pl.ds(0, _RPS_LAST)],
                            den_sh.at[pl.ds(15 * _RPS, _RPS_LAST)])

    plsc.subcore_barrier()

    hbase = c * _HH

    def idx_loads(b, slot):
        return (
            pltpu.make_async_copy(src_hbm.at[s, b], idx_s_[slot],
                                  sem_is[slot]),
            pltpu.make_async_copy(dst_hbm.at[s, b], idx_d_[slot],
                                  sem_id[slot]),
        )

    def gathers(b, slot):
        return (
            pltpu.make_async_copy(el_hbm.at[idx_s_[slot]], elg_[slot],
                                  sem_el[slot]),
            pltpu.make_async_copy(er_hbm.at[idx_d_[slot]], erg_[slot],
                                  sem_er[slot]),
            pltpu.make_async_copy(feat2_hbm.at[idx2_[slot]], rows_[slot],
                                  sem_f[slot]),
        )

    def start_idx(b, slot):
        for cp in idx_loads(b, slot):
            cp.start()

    def issue(b, slot):
        for cp in idx_loads(b, slot):
            cp.wait()

        @plsc.parallel_loop(0, _B // 16, unroll=5)
        def _mk(k):
            v = idx_s_[slot][pl.ds(k * 16, 16)]
            idx2_[slot][pl.ds(k * 16, 16)] = v * 2 + c
        for cp in gathers(b, slot):
            cp.start()

    def process(b, slot):
        cp_el, cp_er, cp_f = gathers(b, slot)
        cp_el.wait()
        cp_er.wait()

        elg = elg_[slot]
        erg = erg_[slot]
        eev = eev_[slot]
        eef = eef_[slot]
        rows = rows_[slot]
        sidx = sidx_[slot]
        idx_d = idx_d_[slot]

        @plsc.parallel_loop(0, _B, unroll=4)
        def _ecomp(k):
            a = elg[k, :] + erg[k, :]
            a = jnp.maximum(a, a * 0.2)
            a = jnp.exp(a)
            eev[k, :] = a
            eef[pl.ds(k * 16, 16)] = a

        @plsc.parallel_loop(0, _B // 16, unroll=5)
        def _icopy(k):
            sidx[pl.ds(k * 16, 16)] = idx_d[pl.ds(k * 16, 16)]

        @pl.when(c == 0)
        def _den_add():
            pltpu.sync_copy(eev, den_sh.at[sidx], add=True)

        cp_f.wait()

        @pl.when(b + 2 < _NBLK)
        def _prefetch_idx():
            start_idx(b + 2, slot)

        @plsc.parallel_loop(0, _B, unroll=2)
        def _scale(e):
            eb = e * _LP + hbase
            for h in range(_HH):
                bidx = jnp.zeros((16,), jnp.int32) + (eb + h)
                bv = plsc.load_gather(eef, [bidx])
                for j2 in range(2):
                    j = (h * 2 + j2) * 16
                    rows[e, pl.ds(j, 16)] = rows[e, pl.ds(j, 16)] * bv

        pltpu.sync_copy(rows, acc_sh.at[sidx], add=True)

    start_idx(0, 0)
    issue(0, 0)
    start_idx(1, 1)

    def pair(g2, carry):
        b0 = 2 * g2
        b1 = b0 + 1
        issue(b1, 1)
        process(b0, 0)

        @pl.when(b0 + 2 < _NBLK)
        def _gather0():
            issue(b0 + 2, 0)

        process(b1, 1)
        return carry

    lax.fori_loop(0, _NBLK // 2, pair, 0)
    plsc.subcore_barrier()

    @pl.when(s < 15)
    def _out_a():
        pltpu.sync_copy(acc_sh.at[pl.ds(s * _RPS, _RPS)],
                        acc_hbm.at[pl.ds(c * _N + s * _RPS, _RPS)])

        @pl.when(c == 0)
        def _dout_a():
            pltpu.sync_copy(den_sh.at[pl.ds(s * _RPS, _RPS)],
                            den_hbm.at[pl.ds(s * _RPS, _RPS)])

    @pl.when(s == 15)
    def _out_b():
        pltpu.sync_copy(acc_sh.at[pl.ds(15 * _RPS, _RPS_LAST)],
                        acc_hbm.at[pl.ds(c * _N + 15 * _RPS, _RPS_LAST)])

        @pl.when(c == 0)
        def _dout_b():
            pltpu.sync_copy(den_sh.at[pl.ds(15 * _RPS, _RPS_LAST)],
                            den_hbm.at[pl.ds(15 * _RPS, _RPS_LAST)])


def _edge_phase(src, dst, feat2, el, er):
    zrow = jnp.zeros((_RPS, _HALF), jnp.float32)
    zden = jnp.zeros((_RPS, _LP), jnp.float32)
    mesh = plsc.VectorSubcoreMesh(core_axis_name="c", subcore_axis_name="s")
    call = pl.kernel(
        _sc_body,
        out_type=[
            jax.ShapeDtypeStruct((_NC * _N, _HALF), jnp.float32),
            jax.ShapeDtypeStruct((_N, _LP), jnp.float32),
        ],
        mesh=mesh,
        compiler_params=pltpu.CompilerParams(
            needs_layout_passes=False, use_tc_tiling_on_sc=False),
        scratch_types=[
            pltpu.VMEM((_B,), jnp.int32),
            pltpu.VMEM((_B,), jnp.int32),
            pltpu.VMEM((_B,), jnp.int32),
            pltpu.VMEM((_B,), jnp.int32),
            pltpu.VMEM((_B,), jnp.int32),
            pltpu.VMEM((_B,), jnp.int32),
            pltpu.VMEM((_B,), jnp.int32),
            pltpu.VMEM((_B,), jnp.int32),
            pltpu.VMEM((_B, _LP), jnp.float32),
            pltpu.VMEM((_B, _LP), jnp.float32),
            pltpu.VMEM((_B, _LP), jnp.float32),
            pltpu.VMEM((_B, _LP), jnp.float32),
            pltpu.VMEM((_B, _LP), jnp.float32),
            pltpu.VMEM((_B, _LP), jnp.float32),
            pltpu.VMEM((_B * _LP,), jnp.float32),
            pltpu.VMEM((_B * _LP,), jnp.float32),
            pltpu.VMEM((_B, _HALF), jnp.float32),
            pltpu.VMEM((_B, _HALF), jnp.float32),
            pltpu.SemaphoreType.DMA,
            pltpu.SemaphoreType.DMA,
            pltpu.SemaphoreType.DMA,
            pltpu.SemaphoreType.DMA,
            pltpu.SemaphoreType.DMA,
            pltpu.SemaphoreType.DMA,
            pltpu.SemaphoreType.DMA,
            pltpu.SemaphoreType.DMA,
            pltpu.SemaphoreType.DMA,
            pltpu.SemaphoreType.DMA,
            pltpu.VMEM_SHARED((_N, _HALF), jnp.float32),
            pltpu.VMEM_SHARED((_N, _LP), jnp.float32),
        ],
    )
    return call(src, dst, feat2, el, er, zrow, zden)


def _epi_body(lo_ref, hi_ref, den_ref, bias_ref, out_ref):
    cat = jnp.concatenate([lo_ref[...], hi_ref[...]], axis=1)
    den = jnp.maximum(den_ref[...][:, :_H], 1e-9)
    den_exp = jnp.broadcast_to(den[:, :, None], (_BN, _H, _D))
    out_ref[...] = cat / den_exp.reshape(_BN, _HD) + bias_ref[...][None, :]


def _epilogue(acc, den, bias):
    grid = _N // _BN
    nb = _N // _BN
    return pl.pallas_call(
        _epi_body,
        grid=(grid,),
        in_specs=[
            pl.BlockSpec((_BN, _HALF), lambda i: (i, 0)),
            pl.BlockSpec((_BN, _HALF), lambda i, _nb=nb: (i + _nb, 0)),
            pl.BlockSpec((_BN, _LP), lambda i: (i, 0)),
            pl.BlockSpec((_HD,), lambda i: (0,)),
        ],
        out_specs=pl.BlockSpec((_BN, _HD), lambda i: (i, 0)),
        out_shape=jax.ShapeDtypeStruct((_N, _HD), jnp.float32),
    )(acc, acc, den, bias)


def kernel(connectivity, nodes_data, W, attn_l, attn_r, bias):
    src = connectivity[0, 0].reshape(_NS, _NBLK, _B)
    dst = connectivity[1, 0].reshape(_NS, _NBLK, _B)
    feat, el, er = _project(nodes_data, W, attn_l, attn_r)
    feat2 = feat.reshape(_NC * _N, _HALF)
    acc, den = _edge_phase(src, dst, feat2, el, er)
    return _epilogue(acc, den, bias)


# speedup vs baseline: 92.4950x; 1.0006x over previous
import functools

import jax
import jax.numpy as jnp
from jax import lax
from jax.experimental import pallas as pl
from jax.experimental.pallas import tpu as pltpu
from jax.experimental.pallas import tpu_sc as plsc

_N = 10000
_IN = 256
_H = 8
_D = 32
_E = 320000
_HD = _H * _D
_HALF = _HD // 2
_HH = _H // 2
_LP = 16

_BN = 400
_B = 80
_NS = 16
_NC = 2
_CHUNK = _E // _NS
_NBLK = _CHUNK // _B
_RPS = 632
_RPS_LAST = _N - 15 * _RPS


def _proj_body(x_ref, w_ref, al_ref, ar_ref, feat_ref, el_ref, er_ref):
    x = x_ref[...]
    w = w_ref[...]
    feat = jnp.dot(x, w, preferred_element_type=jnp.float32)
    feat_ref[...] = feat
    f3 = feat.reshape(_BN, _H, _D)
    el = jnp.sum(f3 * al_ref[...][None, :, :], axis=-1)
    er = jnp.sum(f3 * ar_ref[...][None, :, :], axis=-1)
    pad = jnp.zeros((_BN, _LP - _H), jnp.float32)
    el_ref[...] = jnp.concatenate([el, pad], axis=1)
    er_ref[...] = jnp.concatenate([er, pad], axis=1)


def _project(x, w, al, ar):
    grid = _N // _BN
    return pl.pallas_call(
        _proj_body,
        grid=(grid,),
        in_specs=[
            pl.BlockSpec((_BN, _IN), lambda i: (i, 0)),
            pl.BlockSpec((_IN, _HD), lambda i: (0, 0)),
            pl.BlockSpec((_H, _D), lambda i: (0, 0)),
            pl.BlockSpec((_H, _D), lambda i: (0, 0)),
        ],
        out_specs=[
            pl.BlockSpec((_BN, _HD), lambda i: (i, 0)),
            pl.BlockSpec((_BN, _LP), lambda i: (i, 0)),
            pl.BlockSpec((_BN, _LP), lambda i: (i, 0)),
        ],
        out_shape=[
            jax.ShapeDtypeStruct((_N, _HD), jnp.float32),
            jax.ShapeDtypeStruct((_N, _LP), jnp.float32),
            jax.ShapeDtypeStruct((_N, _LP), jnp.float32),
        ],
    )(x, w, al, ar)


def _sc_body(src_hbm, dst_hbm, feat2_hbm, el_hbm, er_hbm, zrow_hbm, zden_hbm,
             acc_hbm, den_hbm,
             idx_sa, idx_sb, idx_da, idx_db, sidxa, sidxb,
             idx2a, idx2b, elga, elgb, erga, ergb, eeva, eevb, eefa, eefb,
             rowsa, rowsb,
             sem_is_a, sem_is_b, sem_id_a, sem_id_b,
             sem_el_a, sem_el_b, sem_er_a, sem_er_b, sem_f_a, sem_f_b,
             acc_sh, den_sh):
    c = lax.axis_index("c")
    s = lax.axis_index("s")
    idx_s_ = (idx_sa, idx_sb)
    idx_d_ = (idx_da, idx_db)
    sidx_ = (sidxa, sidxb)
    sem_is = (sem_is_a, sem_is_b)
    sem_id = (sem_id_a, sem_id_b)
    idx2_ = (idx2a, idx2b)
    elg_ = (elga, elgb)
    erg_ = (erga, ergb)
    eev_ = (eeva, eevb)
    eef_ = (eefa, eefb)
    rows_ = (rowsa, rowsb)
    sem_el = (sem_el_a, sem_el_b)
    sem_er = (sem_er_a, sem_er_b)
    sem_f = (sem_f_a, sem_f_b)

    @pl.when(s < 15)
    def _zero_a():
        pltpu.sync_copy(zrow_hbm, acc_sh.at[pl.ds(s * _RPS, _RPS)])

        @pl.when(c == 0)
        def _zden_a():
            pltpu.sync_copy(zden_hbm, den_sh.at[pl.ds(s * _RPS, _RPS)])

    @pl.when(s == 15)
    def _zero_b():
        pltpu.sync_copy(zrow_hbm.at[pl.ds(0, _RPS_LAST)],
                        acc_sh.at[pl.ds(15 * _RPS, _RPS_LAST)])

        @pl.when(c == 0)
        def _zden_b():
            pltpu.sync_copy(zden_hbm.at[pl.ds(0, _RPS_LAST)],
                            den_sh.at[pl.ds(15 * _RPS, _RPS_LAST)])

    plsc.subcore_barrier()

    hbase = c * _HH

    def idx_loads(b, slot):
        return (
            pltpu.make_async_copy(src_hbm.at[s, b], idx_s_[slot],
                                  sem_is[slot]),
            pltpu.make_async_copy(dst_hbm.at[s, b], idx_d_[slot],
                                  sem_id[slot]),
        )

    def gathers(b, slot):
        return (
            pltpu.make_async_copy(el_hbm.at[idx_s_[slot]], elg_[slot],
                                  sem_el[slot]),
            pltpu.make_async_copy(er_hbm.at[idx_d_[slot]], erg_[slot],
                                  sem_er[slot]),
            pltpu.make_async_copy(feat2_hbm.at[idx2_[slot]], rows_[slot],
                                  sem_f[slot]),
        )

    def start_idx(b, slot):
        for cp in idx_loads(b, slot):
            cp.start()

    def issue(b, slot):
        for cp in idx_loads(b, slot):
            cp.wait()

        @plsc.parallel_loop(0, _B // 16, unroll=5)
        def _mk(k):
            v = idx_s_[slot][pl.ds(k * 16, 16)]
            idx2_[slot][pl.ds(k * 16, 16)] = v * 2 + c
        for cp in gathers(b, slot):
            cp.start()

    def process(b, slot):
        cp_el, cp_er, cp_f = gathers(b, slot)
        cp_el.wait()
        cp_er.wait()

        elg = elg_[slot]
        erg = erg_[slot]
        eev = eev_[slot]
        eef = eef_[slot]
        rows = rows_[slot]
        sidx = sidx_[slot]
        idx_d = idx_d_[slot]

        @plsc.parallel_loop(0, _B, unroll=8)
        def _ecomp(k):
            a = elg[k, :] + erg[k, :]
            a = jnp.maximum(a, a * 0.2)
            a = jnp.exp(a)
            eev[k, :] = a
            eef[pl.ds(k * 16, 16)] = a

        @plsc.parallel_loop(0, _B // 16, unroll=5)
        def _icopy(k):
            sidx[pl.ds(k * 16, 16)] = idx_d[pl.ds(k * 16, 16)]

        @pl.when(c == 0)
        def _den_add():
            pltpu.sync_copy(eev, den_sh.at[sidx], add=True)

        cp_f.wait()

        @pl.when(b + 2 < _NBLK)
        def _prefetch_idx():
            start_idx(b + 2, slot)

        @plsc.parallel_loop(0, _B, unroll=4)
        def _scale(e):
            eb = e * _LP + hbase
            for h in range(_HH):
                bidx = jnp.zeros((16,), jnp.int32) + (eb + h)
                bv = plsc.load_gather(eef, [bidx])
                for j2 in range(2):
                    j = (h * 2 + j2) * 16
                    rows[e, pl.ds(j, 16)] = rows[e, pl.ds(j, 16)] * bv

        pltpu.sync_copy(rows, acc_sh.at[sidx], add=True)

    start_idx(0, 0)
    issue(0, 0)
    start_idx(1, 1)

    def pair(g2, carry):
        b0 = 2 * g2
        b1 = b0 + 1
        issue(b1, 1)
        process(b0, 0)

        @pl.when(b0 + 2 < _NBLK)
        def _gather0():
            issue(b0 + 2, 0)

        process(b1, 1)
        return carry

    lax.fori_loop(0, _NBLK // 2, pair, 0)
    plsc.subcore_barrier()

    @pl.when(s < 15)
    def _out_a():
        pltpu.sync_copy(acc_sh.at[pl.ds(s * _RPS, _RPS)],
                        acc_hbm.at[pl.ds(c * _N + s * _RPS, _RPS)])

        @pl.when(c == 0)
        def _dout_a():
            pltpu.sync_copy(den_sh.at[pl.ds(s * _RPS, _RPS)],
                            den_hbm.at[pl.ds(s * _RPS, _RPS)])

    @pl.when(s == 15)
    def _out_b():
        pltpu.sync_copy(acc_sh.at[pl.ds(15 * _RPS, _RPS_LAST)],
                        acc_hbm.at[pl.ds(c * _N + 15 * _RPS, _RPS_LAST)])

        @pl.when(c == 0)
        def _dout_b():
            pltpu.sync_copy(den_sh.at[pl.ds(15 * _RPS, _RPS_LAST)],
                            den_hbm.at[pl.ds(15 * _RPS, _RPS_LAST)])


def _edge_phase(src, dst, feat2, el, er):
    zrow = jnp.zeros((_RPS, _HALF), jnp.float32)
    zden = jnp.zeros((_RPS, _LP), jnp.float32)
    mesh = plsc.VectorSubcoreMesh(core_axis_name="c", subcore_axis_name="s")
    call = pl.kernel(
        _sc_body,
        out_type=[
            jax.ShapeDtypeStruct((_NC * _N, _HALF), jnp.float32),
            jax.ShapeDtypeStruct((_N, _LP), jnp.float32),
        ],
        mesh=mesh,
        compiler_params=pltpu.CompilerParams(
            needs_layout_passes=False, use_tc_tiling_on_sc=False),
        scratch_types=[
            pltpu.VMEM((_B,), jnp.int32),
            pltpu.VMEM((_B,), jnp.int32),
            pltpu.VMEM((_B,), jnp.int32),
            pltpu.VMEM((_B,), jnp.int32),
            pltpu.VMEM((_B,), jnp.int32),
            pltpu.VMEM((_B,), jnp.int32),
            pltpu.VMEM((_B,), jnp.int32),
            pltpu.VMEM((_B,), jnp.int32),
            pltpu.VMEM((_B, _LP), jnp.float32),
            pltpu.VMEM((_B, _LP), jnp.float32),
            pltpu.VMEM((_B, _LP), jnp.float32),
            pltpu.VMEM((_B, _LP), jnp.float32),
            pltpu.VMEM((_B, _LP), jnp.float32),
            pltpu.VMEM((_B, _LP), jnp.float32),
            pltpu.VMEM((_B * _LP,), jnp.float32),
            pltpu.VMEM((_B * _LP,), jnp.float32),
            pltpu.VMEM((_B, _HALF), jnp.float32),
            pltpu.VMEM((_B, _HALF), jnp.float32),
            pltpu.SemaphoreType.DMA,
            pltpu.SemaphoreType.DMA,
            pltpu.SemaphoreType.DMA,
            pltpu.SemaphoreType.DMA,
            pltpu.SemaphoreType.DMA,
            pltpu.SemaphoreType.DMA,
            pltpu.SemaphoreType.DMA,
            pltpu.SemaphoreType.DMA,
            pltpu.SemaphoreType.DMA,
            pltpu.SemaphoreType.DMA,
            pltpu.VMEM_SHARED((_N, _HALF), jnp.float32),
            pltpu.VMEM_SHARED((_N, _LP), jnp.float32),
        ],
    )
    return call(src, dst, feat2, el, er, zrow, zden)


def _epi_body(lo_ref, hi_ref, den_ref, bias_ref, out_ref):
    cat = jnp.concatenate([lo_ref[...], hi_ref[...]], axis=1)
    den = jnp.maximum(den_ref[...][:, :_H], 1e-9)
    den_exp = jnp.broadcast_to(den[:, :, None], (_BN, _H, _D))
    out_ref[...] = cat / den_exp.reshape(_BN, _HD) + bias_ref[...][None, :]


def _epilogue(acc, den, bias):
    grid = _N // _BN
    nb = _N // _BN
    return pl.pallas_call(
        _epi_body,
        grid=(grid,),
        in_specs=[
            pl.BlockSpec((_BN, _HALF), lambda i: (i, 0)),
            pl.BlockSpec((_BN, _HALF), lambda i, _nb=nb: (i + _nb, 0)),
            pl.BlockSpec((_BN, _LP), lambda i: (i, 0)),
            pl.BlockSpec((_HD,), lambda i: (0,)),
        ],
        out_specs=pl.BlockSpec((_BN, _HD), lambda i: (i, 0)),
        out_shape=jax.ShapeDtypeStruct((_N, _HD), jnp.float32),
    )(acc, acc, den, bias)


def kernel(connectivity, nodes_data, W, attn_l, attn_r, bias):
    src = connectivity[0, 0].reshape(_NS, _NBLK, _B)
    dst = connectivity[1, 0].reshape(_NS, _NBLK, _B)
    feat, el, er = _project(nodes_data, W, attn_l, attn_r)
    feat2 = feat.reshape(_NC * _N, _HALF)
    acc, den = _edge_phase(src, dst, feat2, el, er)
    return _epilogue(acc, den, bias)
